# Initial kernel scaffold; baseline (speedup 1.0000x reference)
import jax
import jax.numpy as jnp
from jax.experimental import pallas as pl


def kernel(buy_src, buy_dst, cart_src, cart_dst, pv_src, pv_dst, e_type, user_emb, item_emb, edges_emb, W1, W2, W3, W4, EW1, EW2, EW3, EW4):
    raise NotImplementedError("write your pallas kernel here")


# speedup vs baseline: 1.6007x; 1.6007x over previous
import jax
import jax.numpy as jnp
from jax.experimental import pallas as pl

_ROWS = 1000
_NEG_SLOPE = 0.01
_W_BUY = 1.0 / 6.0
_W_CART = 5.0 / 6.0


def _lrelu(x):
    return jnp.where(x >= 0, x, _NEG_SLOPE * x)


def _prep_body(emb_ref, sb_ref, sc_ref, hb_ref, hc_ref):
    e = emb_ref[...] * 0.01
    hb_ref[...] = e * sb_ref[...]
    hc_ref[...] = e * sc_ref[...]


def _mid_body(ab_ref, ac_ref, wb_ref, wc_ref, sb_ref, sc_ref,
              s_ref, hb_ref, hc_ref):
    sb = sb_ref[...]
    sc = sc_ref[...]
    rb = jnp.dot(ab_ref[...], wb_ref[...],
                 preferred_element_type=jnp.float32) * sb
    rc = jnp.dot(ac_ref[...], wc_ref[...],
                 preferred_element_type=jnp.float32) * sc
    s = _W_BUY * _lrelu(rb) + _W_CART * _lrelu(rc)
    s_ref[...] = s
    hb_ref[...] = s * sb
    hc_ref[...] = s * sc


def _last_body(ab_ref, ac_ref, wb_ref, wc_ref, sb_ref, sc_ref, s_ref):
    rb = jnp.dot(ab_ref[...], wb_ref[...],
                 preferred_element_type=jnp.float32) * sb_ref[...]
    rc = jnp.dot(ac_ref[...], wc_ref[...],
                 preferred_element_type=jnp.float32) * sc_ref[...]
    s_ref[...] = _W_BUY * _lrelu(rb) + _W_CART * _lrelu(rc)


def _prep(emb, sb, sc):
    n, d = emb.shape
    row_spec = pl.BlockSpec((_ROWS, d), lambda i: (i, 0))
    s_spec = pl.BlockSpec((_ROWS, 1), lambda i: (i, 0))
    out = jax.ShapeDtypeStruct((n, d), jnp.float32)
    return pl.pallas_call(
        _prep_body,
        grid=(n // _ROWS,),
        in_specs=[row_spec, s_spec, s_spec],
        out_specs=[row_spec, row_spec],
        out_shape=[out, out],
    )(emb, sb, sc)


def _dense_stage(agg_b, agg_c, w_b, w_c, scale_b, scale_c, want_h):
    n, d = agg_b.shape
    row_spec = pl.BlockSpec((_ROWS, d), lambda i: (i, 0))
    s_spec = pl.BlockSpec((_ROWS, 1), lambda i: (i, 0))
    w_spec = pl.BlockSpec((d, d), lambda i: (0, 0))
    in_specs = [row_spec, row_spec, w_spec, w_spec, s_spec, s_spec]
    out = jax.ShapeDtypeStruct((n, d), jnp.float32)
    args = (agg_b, agg_c, w_b, w_c, scale_b, scale_c)
    if want_h:
        return pl.pallas_call(
            _mid_body,
            grid=(n // _ROWS,),
            in_specs=in_specs,
            out_specs=[row_spec, row_spec, row_spec],
            out_shape=[out, out, out],
        )(*args)
    return pl.pallas_call(
        _last_body,
        grid=(n // _ROWS,),
        in_specs=in_specs,
        out_specs=row_spec,
        out_shape=out,
    )(*args)


def kernel(buy_src, buy_dst, cart_src, cart_dst, pv_src, pv_dst, e_type,
           user_emb, item_emb, edges_emb, W1, W2, W3, W4, EW1, EW2, EW3, EW4):
    nu, d = user_emb.shape
    ni = item_emb.shape[0]
    e = buy_src.shape[0]
    ones = jnp.ones((e,), jnp.float32)
    seg = jax.ops.segment_sum

    def inv_sqrt_deg(idx, n):
        return (jnp.maximum(seg(ones, idx, num_segments=n), 1.0) ** -0.5)[:, None]

    su_b = inv_sqrt_deg(buy_src, nu)
    si_b = inv_sqrt_deg(buy_dst, ni)
    su_c = inv_sqrt_deg(cart_src, nu)
    si_c = inv_sqrt_deg(cart_dst, ni)

    ef = edges_emb[e_type] * 0.01
    vb, vc = ef[0], ef[2]
    Ws = (W1, W2, W3, W4)
    EWs = (EW1, EW2, EW3)

    h_ub, h_uc = _prep(user_emb, su_b, su_c)
    h_ib, h_ic = _prep(item_emb, si_b, si_c)

    sf = df = None
    for l in range(4):
        w_eff_b = vb[:, None] * Ws[l]
        w_eff_c = vc[:, None] * Ws[l]
        agg_bd = seg(h_ub[buy_src], buy_dst, num_segments=ni)
        agg_cd = seg(h_uc[cart_src], cart_dst, num_segments=ni)
        agg_bs = seg(h_ib[buy_dst], buy_src, num_segments=nu)
        agg_cs = seg(h_ic[cart_dst], cart_src, num_segments=nu)
        if l == 3:
            df = _dense_stage(agg_bd, agg_cd, w_eff_b, w_eff_c, si_b, si_c, False)
            sf = _dense_stage(agg_bs, agg_cs, w_eff_b, w_eff_c, su_b, su_c, False)
        else:
            df, h_ib, h_ic = _dense_stage(agg_bd, agg_cd, w_eff_b, w_eff_c,
                                          si_b, si_c, True)
            sf, h_ub, h_uc = _dense_stage(agg_bs, agg_cs, w_eff_b, w_eff_c,
                                          su_b, su_c, True)
            vb, vc = vb @ EWs[l], vc @ EWs[l]
    return sf, df
